# Initial kernel scaffold; baseline (speedup 1.0000x reference)
import jax
import jax.numpy as jnp
from jax.experimental import pallas as pl


def kernel(x, edge_index, W, b):
    raise NotImplementedError("write your pallas kernel here")


# speedup vs baseline: 17.1920x; 17.1920x over previous
import functools

import jax
import jax.numpy as jnp
from jax import lax
from jax.experimental import pallas as pl
from jax.experimental.pallas import tpu as pltpu
from jax.experimental.pallas import tpu_sc as plsc

N_NODES = 10000
N_PAD = 10112
D = 128
DH = D // 2
E = 320000
NC = 2
NS = 16
NW = NC * NS
K = 80
DEG_EDGES_PER_W = E // NW
DEG_CHUNKS = DEG_EDGES_PER_W // K
AGG_EDGES_PER_T = E // NS
AGG_CHUNKS = AGG_EDGES_PER_T // K
RPT = N_PAD // NS

_mesh = plsc.VectorSubcoreMesh(core_axis_name="c", subcore_axis_name="s")


def _deg_body(src_hbm, dst_hbm, out_hbm, sidx_v, didx_v, ones_v, zbuf_v,
              degs_sh, degd_sh):
    c = lax.axis_index("c")
    s = lax.axis_index("s")
    wid = s * NC + c
    pltpu.sync_copy(src_hbm.at[wid], sidx_v)
    pltpu.sync_copy(dst_hbm.at[wid], didx_v)

    lane = lax.iota(jnp.int32, 16)
    e0 = jnp.where(lane == 0, 1.0, 0.0).astype(jnp.float32)
    z16 = jnp.zeros((16,), jnp.float32)

    def fill(r, _):
        ones_v[r, :] = e0
        zbuf_v[r, :] = z16
        return 0

    lax.fori_loop(0, RPT, fill, 0)

    pltpu.sync_copy(zbuf_v, degs_sh.at[pl.ds(s * RPT, RPT)])
    pltpu.sync_copy(zbuf_v, degd_sh.at[pl.ds(s * RPT, RPT)])
    plsc.subcore_barrier()

    def body(j, _):
        pltpu.sync_copy(ones_v.at[pl.ds(0, K)], degs_sh.at[sidx_v.at[j]], add=True)
        pltpu.sync_copy(ones_v.at[pl.ds(0, K)], degd_sh.at[didx_v.at[j]], add=True)
        return 0

    lax.fori_loop(0, DEG_CHUNKS, body, 0)
    plsc.subcore_barrier()

    pltpu.sync_copy(degs_sh.at[pl.ds(s * RPT, RPT)],
                    out_hbm.at[c, 0, pl.ds(s * RPT, RPT)])
    pltpu.sync_copy(degd_sh.at[pl.ds(s * RPT, RPT)],
                    out_hbm.at[c, 1, pl.ds(s * RPT, RPT)])


_deg_call = pl.kernel(
    _deg_body,
    out_type=jax.ShapeDtypeStruct((NC, 2, N_PAD, 16), jnp.float32),
    mesh=_mesh,
    scratch_types=[
        pltpu.VMEM((DEG_CHUNKS, K), jnp.int32),
        pltpu.VMEM((DEG_CHUNKS, K), jnp.int32),
        pltpu.VMEM((RPT, 16), jnp.float32),
        pltpu.VMEM((RPT, 16), jnp.float32),
        pltpu.VMEM_SHARED((N_PAD, 16), jnp.float32),
        pltpu.VMEM_SHARED((N_PAD, 16), jnp.float32),
    ],
    compiler_params=pltpu.CompilerParams(use_tc_tiling_on_sc=False),
)


def _agg_body(h_hbm, src_hbm, dst_hbm, out_hbm, sidx_v, didx_v, rows_v,
              acc_sh, sem):
    c = lax.axis_index("c")
    s = lax.axis_index("s")
    pltpu.sync_copy(src_hbm.at[s], sidx_v)
    pltpu.sync_copy(dst_hbm.at[s], didx_v)

    z16 = jnp.zeros((16,), jnp.float32)

    def zfill(r, _):
        for cc in range(DH // 16):
            rows_v[r, pl.ds(cc * 16, 16)] = z16
        return 0

    lax.fori_loop(0, K, zfill, 0)

    for t in range(7):
        pltpu.sync_copy(rows_v, acc_sh.at[pl.ds(s * RPT + t * K, K)])
    pltpu.sync_copy(rows_v.at[pl.ds(0, RPT - 7 * K)],
                    acc_sh.at[pl.ds(s * RPT + 7 * K, RPT - 7 * K)])
    plsc.subcore_barrier()

    def body(j, _):
        pltpu.async_copy(h_hbm.at[c].at[sidx_v.at[j]], rows_v, sem).wait()
        pltpu.sync_copy(rows_v, acc_sh.at[didx_v.at[j]], add=True)
        return 0

    lax.fori_loop(0, AGG_CHUNKS, body, 0)
    plsc.subcore_barrier()

    pltpu.sync_copy(acc_sh.at[pl.ds(s * RPT, RPT)],
                    out_hbm.at[c, pl.ds(s * RPT, RPT)])


_agg_call = pl.kernel(
    _agg_body,
    out_type=jax.ShapeDtypeStruct((NC, N_PAD, DH), jnp.float32),
    mesh=_mesh,
    scratch_types=[
        pltpu.VMEM((AGG_CHUNKS, K), jnp.int32),
        pltpu.VMEM((AGG_CHUNKS, K), jnp.int32),
        pltpu.VMEM((K, DH), jnp.float32),
        pltpu.VMEM_SHARED((N_PAD, DH), jnp.float32),
        pltpu.SemaphoreType.DMA,
    ],
    compiler_params=pltpu.CompilerParams(use_tc_tiling_on_sc=False),
)

_RB = 1000


def _h_body(x_ref, w_ref, b_ref, dp_ref, o_ref):
    dp = dp_ref[...]
    deg = dp[0, 0] + dp[1, 0]
    scale = lax.rsqrt(jnp.sum(deg, axis=1, keepdims=True) + 1e-5)
    h = jnp.dot(x_ref[...], w_ref[0], preferred_element_type=jnp.float32)
    o_ref[0] = (h + b_ref[0]) * scale


def _h_call(x, W2, b2, degparts):
    return pl.pallas_call(
        _h_body,
        grid=(NC, N_NODES // _RB),
        in_specs=[
            pl.BlockSpec((_RB, D), lambda j, i: (i, 0)),
            pl.BlockSpec((1, D, DH), lambda j, i: (j, 0, 0)),
            pl.BlockSpec((1, 1, DH), lambda j, i: (j, 0, 0)),
            pl.BlockSpec((NC, 1, _RB, 16), lambda j, i: (0, 0, i, 0)),
        ],
        out_specs=pl.BlockSpec((1, _RB, DH), lambda j, i: (j, i, 0)),
        out_shape=jax.ShapeDtypeStruct((NC, N_NODES, DH), jnp.float32),
    )(x, W2, b2, degparts)


def _out_body(ap_ref, dp_ref, o_ref):
    ap = ap_ref[...]
    dp = dp_ref[...]
    deg = dp[0, 0] + dp[1, 0]
    dinv = lax.rsqrt(jnp.sum(deg, axis=1, keepdims=True) + 1e-5)
    o_ref[...] = jnp.concatenate([ap[0], ap[1]], axis=-1) * dinv


def _out_call(accparts, degparts):
    return pl.pallas_call(
        _out_body,
        grid=(N_NODES // _RB,),
        in_specs=[
            pl.BlockSpec((NC, _RB, DH), lambda i: (0, i, 0)),
            pl.BlockSpec((NC, 1, _RB, 16), lambda i: (0, 1, i, 0)),
        ],
        out_specs=pl.BlockSpec((_RB, D), lambda i: (i, 0)),
        out_shape=jax.ShapeDtypeStruct((N_NODES, D), jnp.float32),
    )(accparts, degparts)


def kernel(x, edge_index, W, b):
    src = edge_index[0].astype(jnp.int32)
    dst = edge_index[1].astype(jnp.int32)
    src_deg = src.reshape(NW, DEG_CHUNKS, K)
    dst_deg = dst.reshape(NW, DEG_CHUNKS, K)
    src_agg = src.reshape(NS, AGG_CHUNKS, K)
    dst_agg = dst.reshape(NS, AGG_CHUNKS, K)
    W2 = W.astype(jnp.float32).reshape(1, D, D)
    W2 = jnp.concatenate([W2[:, :, :DH], W2[:, :, DH:]], axis=0)
    b2 = b.astype(jnp.float32).reshape(1, 1, D)
    b2 = jnp.concatenate([b2[:, :, :DH], b2[:, :, DH:]], axis=0)
    degparts = _deg_call(src_deg, dst_deg)
    hsplit = _h_call(x.astype(jnp.float32), W2, b2, degparts)
    accparts = _agg_call(hsplit, src_agg, dst_agg)
    return _out_call(accparts, degparts)


# speedup vs baseline: 28.2482x; 1.6431x over previous
import functools

import jax
import jax.numpy as jnp
from jax import lax
from jax.experimental import pallas as pl
from jax.experimental.pallas import tpu as pltpu
from jax.experimental.pallas import tpu_sc as plsc

N_NODES = 10000
N_PAD = 10112
D = 128
DH = D // 2
E = 320000
NC = 2
NS = 16
NW = NC * NS
K = 125
DEG_EDGES_PER_W = E // NW
DEG_CHUNKS = DEG_EDGES_PER_W // K
AGG_EDGES_PER_T = E // NS
AGG_CHUNKS = AGG_EDGES_PER_T // K
NBUF = 5
RPT = N_PAD // NS

_mesh = plsc.VectorSubcoreMesh(core_axis_name="c", subcore_axis_name="s")


def _deg_body(src_hbm, dst_hbm, out_hbm, sidx_v, didx_v, ones_v, zbuf_v,
              degs_sh, degd_sh):
    c = lax.axis_index("c")
    s = lax.axis_index("s")
    wid = s * NC + c
    pltpu.sync_copy(src_hbm.at[wid], sidx_v)
    pltpu.sync_copy(dst_hbm.at[wid], didx_v)

    lane = lax.iota(jnp.int32, 16)
    e0 = jnp.where(lane == 0, 1.0, 0.0).astype(jnp.float32)
    z16 = jnp.zeros((16,), jnp.float32)

    def zfill(r, _):
        zbuf_v[r, :] = z16
        return 0

    lax.fori_loop(0, RPT, zfill, 0)

    def ofill(r, _):
        ones_v[r, :] = e0
        return 0

    lax.fori_loop(0, K, ofill, 0)

    pltpu.sync_copy(zbuf_v, degs_sh.at[pl.ds(s * RPT, RPT)])
    pltpu.sync_copy(zbuf_v, degd_sh.at[pl.ds(s * RPT, RPT)])
    plsc.subcore_barrier()

    def body(j, _):
        pltpu.sync_copy(ones_v, degs_sh.at[sidx_v.at[j]], add=True)
        pltpu.sync_copy(ones_v, degd_sh.at[didx_v.at[j]], add=True)
        return 0

    lax.fori_loop(0, DEG_CHUNKS, body, 0)
    plsc.subcore_barrier()

    pltpu.sync_copy(degs_sh.at[pl.ds(s * RPT, RPT)],
                    out_hbm.at[c, 0, pl.ds(s * RPT, RPT)])
    pltpu.sync_copy(degd_sh.at[pl.ds(s * RPT, RPT)],
                    out_hbm.at[c, 1, pl.ds(s * RPT, RPT)])


_deg_call = pl.kernel(
    _deg_body,
    out_type=jax.ShapeDtypeStruct((NC, 2, N_PAD, 16), jnp.float32),
    mesh=_mesh,
    scratch_types=[
        pltpu.VMEM((DEG_CHUNKS, K), jnp.int32),
        pltpu.VMEM((DEG_CHUNKS, K), jnp.int32),
        pltpu.VMEM((K, 16), jnp.float32),
        pltpu.VMEM((RPT, 16), jnp.float32),
        pltpu.VMEM_SHARED((N_PAD, 16), jnp.float32),
        pltpu.VMEM_SHARED((N_PAD, 16), jnp.float32),
    ],
    compiler_params=pltpu.CompilerParams(use_tc_tiling_on_sc=False),
)


def _agg_body(h_hbm, src_hbm, dst_hbm, out_hbm, sidx_v, didx_v, rv,
              acc_sh, gsem, ssem):
    c = lax.axis_index("c")
    s = lax.axis_index("s")
    pltpu.sync_copy(src_hbm.at[s], sidx_v)
    pltpu.sync_copy(dst_hbm.at[s], didx_v)

    z16 = jnp.zeros((16,), jnp.float32)

    def zfill(r, _):
        for cc in range(DH // 16):
            rv[0, r, pl.ds(cc * 16, 16)] = z16
        return 0

    lax.fori_loop(0, K, zfill, 0)

    for t in range(5):
        pltpu.sync_copy(rv.at[0], acc_sh.at[pl.ds(s * RPT + t * K, K)])
    pltpu.sync_copy(rv.at[0].at[pl.ds(0, RPT - 5 * K)],
                    acc_sh.at[pl.ds(s * RPT + 5 * K, RPT - 5 * K)])
    plsc.subcore_barrier()

    for t in range(NBUF):
        pltpu.async_copy(h_hbm.at[c].at[sidx_v.at[t]], rv.at[t], gsem.at[t])

    def body(i, _):
        jj = i * NBUF
        for t in range(NBUF):
            pltpu.make_async_copy(h_hbm.at[c].at[sidx_v.at[jj + t]],
                                  rv.at[t], gsem.at[t]).wait()
        descs = []
        for t in range(NBUF):
            descs.append(pltpu.async_copy(rv.at[t], acc_sh.at[didx_v.at[jj + t]],
                                          ssem.at[t], add=True))
        for t in range(NBUF):
            descs[t].wait()

            @pl.when(jj + NBUF < AGG_CHUNKS)
            def _(t=t, jj=jj):
                pltpu.async_copy(h_hbm.at[c].at[sidx_v.at[jj + t + NBUF]],
                                 rv.at[t], gsem.at[t])

        return 0

    lax.fori_loop(0, AGG_CHUNKS // NBUF, body, 0)
    plsc.subcore_barrier()

    pltpu.sync_copy(acc_sh.at[pl.ds(s * RPT, RPT)],
                    out_hbm.at[c, pl.ds(s * RPT, RPT)])


_agg_call = pl.kernel(
    _agg_body,
    out_type=jax.ShapeDtypeStruct((NC, N_PAD, DH), jnp.float32),
    mesh=_mesh,
    scratch_types=[
        pltpu.VMEM((AGG_CHUNKS, K), jnp.int32),
        pltpu.VMEM((AGG_CHUNKS, K), jnp.int32),
        pltpu.VMEM((NBUF, K, DH), jnp.float32),
        pltpu.VMEM_SHARED((N_PAD, DH), jnp.float32),
        pltpu.SemaphoreType.DMA((NBUF,)),
        pltpu.SemaphoreType.DMA((NBUF,)),
    ],
    compiler_params=pltpu.CompilerParams(use_tc_tiling_on_sc=False),
)

_RB = 1000


def _h_body(x_ref, w_ref, b_ref, dp_ref, o_ref):
    dp = dp_ref[...]
    deg = dp[0, 0] + dp[1, 0]
    scale = lax.rsqrt(jnp.sum(deg, axis=1, keepdims=True) + 1e-5)
    h = jnp.dot(x_ref[...], w_ref[0], preferred_element_type=jnp.float32)
    o_ref[0] = (h + b_ref[0]) * scale


def _h_call(x, W2, b2, degparts):
    return pl.pallas_call(
        _h_body,
        grid=(NC, N_NODES // _RB),
        in_specs=[
            pl.BlockSpec((_RB, D), lambda j, i: (i, 0)),
            pl.BlockSpec((1, D, DH), lambda j, i: (j, 0, 0)),
            pl.BlockSpec((1, 1, DH), lambda j, i: (j, 0, 0)),
            pl.BlockSpec((NC, 1, _RB, 16), lambda j, i: (0, 0, i, 0)),
        ],
        out_specs=pl.BlockSpec((1, _RB, DH), lambda j, i: (j, i, 0)),
        out_shape=jax.ShapeDtypeStruct((NC, N_NODES, DH), jnp.float32),
    )(x, W2, b2, degparts)


def _out_body(ap_ref, dp_ref, o_ref):
    ap = ap_ref[...]
    dp = dp_ref[...]
    deg = dp[0, 0] + dp[1, 0]
    dinv = lax.rsqrt(jnp.sum(deg, axis=1, keepdims=True) + 1e-5)
    o_ref[...] = jnp.concatenate([ap[0], ap[1]], axis=-1) * dinv


def _out_call(accparts, degparts):
    return pl.pallas_call(
        _out_body,
        grid=(N_NODES // _RB,),
        in_specs=[
            pl.BlockSpec((NC, _RB, DH), lambda i: (0, i, 0)),
            pl.BlockSpec((NC, 1, _RB, 16), lambda i: (0, 1, i, 0)),
        ],
        out_specs=pl.BlockSpec((_RB, D), lambda i: (i, 0)),
        out_shape=jax.ShapeDtypeStruct((N_NODES, D), jnp.float32),
    )(accparts, degparts)


def kernel(x, edge_index, W, b):
    src = edge_index[0].astype(jnp.int32)
    dst = edge_index[1].astype(jnp.int32)
    src_deg = src.reshape(NW, DEG_CHUNKS, K)
    dst_deg = dst.reshape(NW, DEG_CHUNKS, K)
    src_agg = src.reshape(NS, AGG_CHUNKS, K)
    dst_agg = dst.reshape(NS, AGG_CHUNKS, K)
    W2 = W.astype(jnp.float32).reshape(1, D, D)
    W2 = jnp.concatenate([W2[:, :, :DH], W2[:, :, DH:]], axis=0)
    b2 = b.astype(jnp.float32).reshape(1, 1, D)
    b2 = jnp.concatenate([b2[:, :, :DH], b2[:, :, DH:]], axis=0)
    degparts = _deg_call(src_deg, dst_deg)
    hsplit = _h_call(x.astype(jnp.float32), W2, b2, degparts)
    accparts = _agg_call(hsplit, src_agg, dst_agg)
    return _out_call(accparts, degparts)


# speedup vs baseline: 29.1267x; 1.0311x over previous
import functools

import jax
import jax.numpy as jnp
from jax import lax
from jax.experimental import pallas as pl
from jax.experimental.pallas import tpu as pltpu
from jax.experimental.pallas import tpu_sc as plsc

N_NODES = 10000
N_PAD = 10112
D = 128
DH = D // 2
E = 320000
NC = 2
NS = 16
NW = NC * NS
K = 125
DEG_EDGES_PER_W = E // NW
DEG_CHUNKS = DEG_EDGES_PER_W // K
AGG_EDGES_PER_T = E // NS
AGG_CHUNKS = AGG_EDGES_PER_T // K
NBUF = 5
RPT = N_PAD // NS

_mesh = plsc.VectorSubcoreMesh(core_axis_name="c", subcore_axis_name="s")


def _deg_body(src_hbm, dst_hbm, out_hbm, sidx_v, didx_v, ones_v, zbuf_v,
              degs_sh, degd_sh, gsem, dsem):
    c = lax.axis_index("c")
    s = lax.axis_index("s")
    wid = s * NC + c
    pltpu.sync_copy(src_hbm.at[wid], sidx_v)
    pltpu.sync_copy(dst_hbm.at[wid], didx_v)

    lane = lax.iota(jnp.int32, 16)
    e0 = jnp.where(lane == 0, 1.0, 0.0).astype(jnp.float32)
    z16 = jnp.zeros((16,), jnp.float32)

    def zfill(r, _):
        zbuf_v[r, :] = z16
        return 0

    lax.fori_loop(0, RPT, zfill, 0)

    def ofill(r, _):
        ones_v[r, :] = e0
        return 0

    lax.fori_loop(0, K, ofill, 0)

    pltpu.sync_copy(zbuf_v, degs_sh.at[pl.ds(s * RPT, RPT)])
    pltpu.sync_copy(zbuf_v, degd_sh.at[pl.ds(s * RPT, RPT)])
    plsc.subcore_barrier()

    def body(i, _):
        jj = i * NBUF
        descs = []
        for t in range(NBUF):
            descs.append(pltpu.async_copy(ones_v, degs_sh.at[sidx_v.at[jj + t]],
                                          gsem.at[t], add=True))
            descs.append(pltpu.async_copy(ones_v, degd_sh.at[didx_v.at[jj + t]],
                                          dsem.at[t], add=True))
        for d in descs:
            d.wait()
        return 0

    lax.fori_loop(0, DEG_CHUNKS // NBUF, body, 0)
    plsc.subcore_barrier()

    pltpu.sync_copy(degs_sh.at[pl.ds(s * RPT, RPT)],
                    out_hbm.at[c, 0, pl.ds(s * RPT, RPT)])
    pltpu.sync_copy(degd_sh.at[pl.ds(s * RPT, RPT)],
                    out_hbm.at[c, 1, pl.ds(s * RPT, RPT)])


_deg_call = pl.kernel(
    _deg_body,
    out_type=jax.ShapeDtypeStruct((NC, 2, N_PAD, 16), jnp.float32),
    mesh=_mesh,
    scratch_types=[
        pltpu.VMEM((DEG_CHUNKS, K), jnp.int32),
        pltpu.VMEM((DEG_CHUNKS, K), jnp.int32),
        pltpu.VMEM((K, 16), jnp.float32),
        pltpu.VMEM((RPT, 16), jnp.float32),
        pltpu.VMEM_SHARED((N_PAD, 16), jnp.float32),
        pltpu.VMEM_SHARED((N_PAD, 16), jnp.float32),
        pltpu.SemaphoreType.DMA((NBUF,)),
        pltpu.SemaphoreType.DMA((NBUF,)),
    ],
    compiler_params=pltpu.CompilerParams(use_tc_tiling_on_sc=False),
)


def _agg_body(h_hbm, src_hbm, dst_hbm, out_hbm, sidx_v, didx_v, rv,
              acc_sh, gsem, ssem):
    c = lax.axis_index("c")
    s = lax.axis_index("s")
    pltpu.sync_copy(src_hbm.at[s], sidx_v)
    pltpu.sync_copy(dst_hbm.at[s], didx_v)

    z16 = jnp.zeros((16,), jnp.float32)

    def zfill(r, _):
        for cc in range(DH // 16):
            rv[0, r, pl.ds(cc * 16, 16)] = z16
        return 0

    lax.fori_loop(0, K, zfill, 0)

    for t in range(5):
        pltpu.sync_copy(rv.at[0], acc_sh.at[pl.ds(s * RPT + t * K, K)])
    pltpu.sync_copy(rv.at[0].at[pl.ds(0, RPT - 5 * K)],
                    acc_sh.at[pl.ds(s * RPT + 5 * K, RPT - 5 * K)])
    plsc.subcore_barrier()

    for t in range(NBUF):
        pltpu.async_copy(h_hbm.at[c].at[sidx_v.at[t]], rv.at[t], gsem.at[t])

    def body(i, _):
        jj = i * NBUF
        for t in range(NBUF):
            pltpu.make_async_copy(h_hbm.at[c].at[sidx_v.at[jj + t]],
                                  rv.at[t], gsem.at[t]).wait()
        descs = []
        for t in range(NBUF):
            descs.append(pltpu.async_copy(rv.at[t], acc_sh.at[didx_v.at[jj + t]],
                                          ssem.at[t], add=True))
        for t in range(NBUF):
            descs[t].wait()

            @pl.when(jj + NBUF < AGG_CHUNKS)
            def _(t=t, jj=jj):
                pltpu.async_copy(h_hbm.at[c].at[sidx_v.at[jj + t + NBUF]],
                                 rv.at[t], gsem.at[t])

        return 0

    lax.fori_loop(0, AGG_CHUNKS // NBUF, body, 0)
    plsc.subcore_barrier()

    pltpu.sync_copy(acc_sh.at[pl.ds(s * RPT, RPT)],
                    out_hbm.at[c, pl.ds(s * RPT, RPT)])


_agg_call = pl.kernel(
    _agg_body,
    out_type=jax.ShapeDtypeStruct((NC, N_PAD, DH), jnp.float32),
    mesh=_mesh,
    scratch_types=[
        pltpu.VMEM((AGG_CHUNKS, K), jnp.int32),
        pltpu.VMEM((AGG_CHUNKS, K), jnp.int32),
        pltpu.VMEM((NBUF, K, DH), jnp.float32),
        pltpu.VMEM_SHARED((N_PAD, DH), jnp.float32),
        pltpu.SemaphoreType.DMA((NBUF,)),
        pltpu.SemaphoreType.DMA((NBUF,)),
    ],
    compiler_params=pltpu.CompilerParams(use_tc_tiling_on_sc=False),
)

_RB = 1000


def _h_body(x_ref, w_ref, b_ref, dp_ref, o_ref):
    dp = dp_ref[...]
    deg = dp[0, 0] + dp[1, 0]
    scale = lax.rsqrt(jnp.sum(deg, axis=1, keepdims=True) + 1e-5)
    h = jnp.dot(x_ref[...], w_ref[0], preferred_element_type=jnp.float32)
    o_ref[0] = (h + b_ref[0]) * scale


def _h_call(x, W2, b2, degparts):
    return pl.pallas_call(
        _h_body,
        grid=(NC, N_NODES // _RB),
        in_specs=[
            pl.BlockSpec((_RB, D), lambda j, i: (i, 0)),
            pl.BlockSpec((1, D, DH), lambda j, i: (j, 0, 0)),
            pl.BlockSpec((1, 1, DH), lambda j, i: (j, 0, 0)),
            pl.BlockSpec((NC, 1, _RB, 16), lambda j, i: (0, 0, i, 0)),
        ],
        out_specs=pl.BlockSpec((1, _RB, DH), lambda j, i: (j, i, 0)),
        out_shape=jax.ShapeDtypeStruct((NC, N_NODES, DH), jnp.float32),
    )(x, W2, b2, degparts)


def _out_body(ap_ref, dp_ref, o_ref):
    ap = ap_ref[...]
    dp = dp_ref[...]
    deg = dp[0, 0] + dp[1, 0]
    dinv = lax.rsqrt(jnp.sum(deg, axis=1, keepdims=True) + 1e-5)
    o_ref[...] = jnp.concatenate([ap[0], ap[1]], axis=-1) * dinv


def _out_call(accparts, degparts):
    return pl.pallas_call(
        _out_body,
        grid=(N_NODES // _RB,),
        in_specs=[
            pl.BlockSpec((NC, _RB, DH), lambda i: (0, i, 0)),
            pl.BlockSpec((NC, 1, _RB, 16), lambda i: (0, 1, i, 0)),
        ],
        out_specs=pl.BlockSpec((_RB, D), lambda i: (i, 0)),
        out_shape=jax.ShapeDtypeStruct((N_NODES, D), jnp.float32),
    )(accparts, degparts)


def kernel(x, edge_index, W, b):
    src = edge_index[0].astype(jnp.int32)
    dst = edge_index[1].astype(jnp.int32)
    src_deg = src.reshape(NW, DEG_CHUNKS, K)
    dst_deg = dst.reshape(NW, DEG_CHUNKS, K)
    src_agg = src.reshape(NS, AGG_CHUNKS, K)
    dst_agg = dst.reshape(NS, AGG_CHUNKS, K)
    W2 = W.astype(jnp.float32).reshape(1, D, D)
    W2 = jnp.concatenate([W2[:, :, :DH], W2[:, :, DH:]], axis=0)
    b2 = b.astype(jnp.float32).reshape(1, 1, D)
    b2 = jnp.concatenate([b2[:, :, :DH], b2[:, :, DH:]], axis=0)
    degparts = _deg_call(src_deg, dst_deg)
    hsplit = _h_call(x.astype(jnp.float32), W2, b2, degparts)
    accparts = _agg_call(hsplit, src_agg, dst_agg)
    return _out_call(accparts, degparts)
